# baseline (speedup 1.0000x reference)
import jax
import jax.numpy as jnp
from jax import lax
from jax.experimental import pallas as pl

B, N, F = 8, 50000, 128
TCHUNK = 2000
NTCHUNK = N // TCHUNK
KSPLIT = 8
SUB = TCHUNK // KSPLIT


def _tc_body(x_ref, o_ref):
    j = pl.program_id(0)

    @pl.when(j == 0)
    def _init():
        o_ref[...] = jnp.zeros_like(o_ref)

    ones = jnp.ones((TCHUNK,), jnp.float32)
    o_ref[...] += lax.dot_general(
        x_ref[...], ones, (((1,), (0,)), ((), ())),
        preferred_element_type=jnp.float32,
    ) * (1.0 / N)


@jax.jit
def kernel(x):
    return pl.pallas_call(
        _tc_body,
        grid=(NTCHUNK,),
        in_specs=[pl.BlockSpec((B, TCHUNK, F), lambda j: (0, j, 0))],
        out_specs=pl.BlockSpec((B, F), lambda j: (0, 0)),
        out_shape=jax.ShapeDtypeStruct((B, F), jnp.float32),
    )(x)
